# Initial kernel scaffold; baseline (speedup 1.0000x reference)
import jax
import jax.numpy as jnp
from jax.experimental import pallas as pl


def kernel(anchors, objectness, pred_bbox_delta):
    raise NotImplementedError("write your pallas kernel here")


# speedup vs baseline: 55.9783x; 55.9783x over previous
import functools

import jax
import jax.numpy as jnp
import numpy as np
from jax.experimental import pallas as pl
from jax.experimental.pallas import tpu as pltpu

N_IN = 20000
N_PAD = 20480
ROWS = 160
LANES = 128
K_SEL = 2000
D = 2048
B = 256
CH = 512
OUT_K = 1000
OUT_PAD = 1024
NMS_THRESH = 0.7
IMG_H, IMG_W = 800.0, 800.0
MIN_SIZE = 1.0
CLIP = float(np.log(1000.0 / 16.0))
NEG = -3.0e38
INT_MIN = -2147483648

_HI = jax.lax.Precision.HIGHEST


def _dot(a, b, dims):
    return jax.lax.dot_general(a, b, (dims, ((), ())),
                               preferred_element_type=jnp.float32,
                               precision=_HI)


def _select_kernel(obj_ref, p_ref):
    obj = obj_ref[:, :]
    raw = jax.lax.bitcast_convert_type(obj, jnp.int32)
    key = jnp.where(raw >= 0, raw, raw ^ jnp.int32(0x7FFFFFFF))

    k = jnp.int32(K_SEL)
    cnt_nonneg = jnp.sum((key >= 0).astype(jnp.int32))
    t0 = jnp.where(cnt_nonneg >= k, jnp.int32(0), jnp.int32(INT_MIN))

    def body(_, carry):
        t, bit = carry
        t2 = t | bit
        cnt = jnp.sum((key >= t2).astype(jnp.int32))
        return jnp.where(cnt >= k, t2, t), jax.lax.shift_right_logical(bit, 1)

    thr, _ = jax.lax.fori_loop(0, 31, body, (t0, jnp.int32(1 << 30)))

    cnt_gt = jnp.sum((key > thr).astype(jnp.int32))
    extra = (k - cnt_gt).astype(jnp.float32)

    lane = jax.lax.broadcasted_iota(jnp.int32, (LANES, LANES), 0)
    laneT = jax.lax.broadcasted_iota(jnp.int32, (LANES, LANES), 1)
    u_strict = (lane < laneT).astype(jnp.float32)
    row = jax.lax.broadcasted_iota(jnp.int32, (ROWS, ROWS), 0)
    rowT = jax.lax.broadcasted_iota(jnp.int32, (ROWS, ROWS), 1)
    l_strict = (rowT < row).astype(jnp.float32)

    def excl_cumsum(mask_f):
        within = _dot(mask_f, u_strict, ((1,), (0,)))
        rowsum = jnp.sum(mask_f, axis=1, keepdims=True)
        offs = _dot(l_strict, rowsum, ((1,), (0,)))
        return within + offs

    eq = (key == thr)
    eq_rank = excl_cumsum(eq.astype(jnp.float32))
    sel = (key > thr) | (eq & (eq_rank < extra))
    pos = excl_cumsum(sel.astype(jnp.float32))
    p_ref[:, :] = jnp.where(sel, pos, jnp.float32(D))


def _decode(sc, a1, a2, a3, a4, d1, d2, d3, d4, real):
    s = jnp.where(real, sc, NEG)
    dw = jnp.minimum(d3, CLIP)
    dh = jnp.minimum(d4, CLIP)
    widths = a3 - a1
    heights = a4 - a2
    ctr_x = a1 + 0.5 * widths
    ctr_y = a2 + 0.5 * heights
    pred_ctr_x = d1 * widths + ctr_x
    pred_ctr_y = d2 * heights + ctr_y
    pred_w = jnp.exp(dw) * widths
    pred_h = jnp.exp(dh) * heights
    x1 = jnp.clip(pred_ctr_x - 0.5 * pred_w, 0.0, IMG_W)
    y1 = jnp.clip(pred_ctr_y - 0.5 * pred_h, 0.0, IMG_H)
    x2 = jnp.clip(pred_ctr_x + 0.5 * pred_w, 0.0, IMG_W)
    y2 = jnp.clip(pred_ctr_y + 0.5 * pred_h, 0.0, IMG_H)
    valid = ((x2 - x1) >= MIN_SIZE) & ((y2 - y1) >= MIN_SIZE)
    ms = jnp.where(valid, s, NEG)
    area = (x2 - x1) * (y2 - y1)
    return x1, y1, x2, y2, area, ms, s


def _propose_kernel(p_ref, xt_ref, out_ref, mt_ref):
    f32 = jnp.float32
    ones11 = jnp.ones((1, 1), f32)
    slot_row = jax.lax.broadcasted_iota(jnp.int32, (1, D), 1).astype(f32)
    eye16 = (jax.lax.broadcasted_iota(jnp.int32, (16, 16), 0)
             == jax.lax.broadcasted_iota(jnp.int32, (16, 16), 1)).astype(f32)

    acc_t = jnp.zeros((16, D), dtype=f32)
    for c in range(N_PAD // CH):
        pc_row = p_ref[0:1, c * CH:(c + 1) * CH]
        pc_col = _dot(pc_row, ones11, ((0,), (0,)))
        onehot = (pc_col == slot_row).astype(f32)
        xc = xt_ref[:, c * CH:(c + 1) * CH]
        acc_t = acc_t + _dot(xc, onehot, ((1,), (0,)))
    acc = _dot(acc_t, eye16, ((0,), (0,)))

    i_col = jax.lax.broadcasted_iota(jnp.int32, (D, 1), 0)
    j_row = jax.lax.broadcasted_iota(jnp.int32, (1, D), 1)
    colv = _decode(acc[:, 0:1], acc[:, 1:2], acc[:, 2:3], acc[:, 3:4],
                   acc[:, 4:5], acc[:, 5:6], acc[:, 6:7], acc[:, 7:8],
                   acc[:, 8:9], i_col < K_SEL)
    rowv = _decode(acc_t[0:1, :], acc_t[1:2, :], acc_t[2:3, :], acc_t[3:4, :],
                   acc_t[4:5, :], acc_t[5:6, :], acc_t[6:7, :], acc_t[7:8, :],
                   acc_t[8:9, :], j_row < K_SEL)
    ms_c, s_c = colv[5], colv[6]
    ms_r, s_r = rowv[5], rowv[6]
    pos_c = i_col.astype(f32)
    pos_r = j_row.astype(f32)

    col8 = jax.lax.broadcasted_iota(jnp.int32, (1, 8), 1)
    P = jnp.zeros((D, 8), dtype=f32)
    for j, v in enumerate(colv + (pos_c,)):
        P = jnp.where(col8 == j, v, P)

    sp = jnp.zeros((D, 8), dtype=f32)
    for ib in range(D // B):
        sl = slice(ib * B, (ib + 1) * B)
        msb, sb, posb = ms_c[sl, :], s_c[sl, :], pos_c[sl, :]
        before = ((ms_r > msb)
                  | ((ms_r == msb)
                     & ((s_r > sb) | ((s_r == sb) & (pos_r < posb)))))
        rank_b = jnp.sum(before.astype(f32), axis=1, keepdims=True)
        perm_b = (rank_b == slot_row).astype(f32)
        sp = sp + _dot(perm_b, P[sl, :], ((0,), (0,)))
    eye8 = eye16[0:8, 0:8]
    spt = _dot(eye8, sp, ((1,), (1,)))

    for ib in range(D // B):
        sl = slice(ib * B, (ib + 1) * B)
        xx1 = jnp.maximum(sp[sl, 0:1], spt[0:1, :])
        yy1 = jnp.maximum(sp[sl, 1:2], spt[1:2, :])
        xx2 = jnp.minimum(sp[sl, 2:3], spt[2:3, :])
        yy2 = jnp.minimum(sp[sl, 3:4], spt[3:4, :])
        inter = jnp.maximum(xx2 - xx1, 0.0) * jnp.maximum(yy2 - yy1, 0.0)
        iou = inter / (sp[sl, 4:5] + spt[4:5, :] - inter + 1e-9)
        mt_ref[sl, :] = ((iou > NMS_THRESH)
                         & (i_col[sl, :] < j_row)).astype(f32)

    def fix_cond(carry):
        _, done = carry
        return jnp.logical_not(done)

    def fix_body(carry):
        supp, _ = carry
        keep = 1.0 - supp
        hit = _dot(keep, mt_ref[:, :], ((1,), (0,)))
        new = (hit > 0.0).astype(f32)
        return new, jnp.all(new == supp)

    supp, _ = jax.lax.while_loop(
        fix_cond, fix_body, (jnp.zeros((1, D), dtype=f32), jnp.bool_(False)))

    fs_r = jnp.where(supp > 0.0, NEG, spt[5:6, :])
    fs_c = _dot(fs_r, ones11, ((0,), (0,)))
    out_acc = jnp.zeros((OUT_PAD, 8), dtype=f32)
    rank_iota = jax.lax.broadcasted_iota(jnp.int32, (1, OUT_PAD), 1).astype(f32)
    for ib in range(D // B):
        sl = slice(ib * B, (ib + 1) * B)
        fb = ((fs_r > fs_c[sl, :])
              | ((fs_r == fs_c[sl, :]) & (pos_r < pos_c[sl, :])))
        frank_b = jnp.sum(fb.astype(f32), axis=1, keepdims=True)
        hb = (frank_b == rank_iota).astype(f32)
        out_acc = out_acc + _dot(hb, sp[sl, :], ((0,), (0,)))
    out_ref[:, :] = out_acc


@functools.partial(jax.jit, static_argnames=("interpret",))
def kernel(anchors, objectness, pred_bbox_delta, interpret=False):
    obj = objectness.astype(jnp.float32)
    obj_pad = jnp.full((N_PAD,), -jnp.inf, dtype=jnp.float32)
    obj_pad = jax.lax.dynamic_update_slice(obj_pad, obj, (0,))
    obj2d = obj_pad.reshape(ROWS, LANES)

    x = jnp.concatenate([obj[:, None], anchors.astype(jnp.float32),
                         pred_bbox_delta.astype(jnp.float32),
                         jnp.zeros((N_IN, 7), jnp.float32)], axis=1)
    x = jnp.pad(x, ((0, N_PAD - N_IN), (0, 0)))
    xt = x.T

    p2d = pl.pallas_call(
        _select_kernel,
        out_shape=jax.ShapeDtypeStruct((ROWS, LANES), jnp.float32),
        interpret=interpret,
    )(obj2d)
    p = p2d.reshape(1, N_PAD)

    out = pl.pallas_call(
        _propose_kernel,
        out_shape=jax.ShapeDtypeStruct((OUT_PAD, 8), jnp.float32),
        scratch_shapes=[pltpu.VMEM((D, D), jnp.float32)],
        interpret=interpret,
    )(p, xt)
    return out[:OUT_K, :4]


# speedup vs baseline: 63.7380x; 1.1386x over previous
import functools

import jax
import jax.numpy as jnp
import numpy as np
from jax.experimental import pallas as pl
from jax.experimental.pallas import tpu as pltpu

N_IN = 20000
N_PAD = 20480
ROWS = 160
LANES = 128
K_SEL = 2000
D = 2048
B = 256
CH = 512
OUT_K = 1000
OUT_PAD = 1024
NMS_THRESH = 0.7
IMG_H, IMG_W = 800.0, 800.0
MIN_SIZE = 1.0
CLIP = float(np.log(1000.0 / 16.0))
NEG = -3.0e38
INT_MIN = -2147483648

_HI = jax.lax.Precision.HIGHEST
_LO = jax.lax.Precision.DEFAULT


def _dot(a, b, dims, prec=_HI):
    return jax.lax.dot_general(a, b, (dims, ((), ())),
                               preferred_element_type=jnp.float32,
                               precision=prec)


def _select_kernel(obj_ref, p_ref):
    obj = obj_ref[:, :]
    raw = jax.lax.bitcast_convert_type(obj, jnp.int32)
    key = jnp.where(raw >= 0, raw, raw ^ jnp.int32(0x7FFFFFFF))

    k = jnp.int32(K_SEL)
    cnt_nonneg = jnp.sum((key >= 0).astype(jnp.int32))
    t0 = jnp.where(cnt_nonneg >= k, jnp.int32(0), jnp.int32(INT_MIN))

    def body(_, carry):
        t, bit = carry
        t2 = t | bit
        cnt = jnp.sum((key >= t2).astype(jnp.int32))
        return jnp.where(cnt >= k, t2, t), jax.lax.shift_right_logical(bit, 1)

    thr, _ = jax.lax.fori_loop(0, 31, body, (t0, jnp.int32(1 << 30)))

    cnt_gt = jnp.sum((key > thr).astype(jnp.int32))
    extra = (k - cnt_gt).astype(jnp.float32)

    lane = jax.lax.broadcasted_iota(jnp.int32, (LANES, LANES), 0)
    laneT = jax.lax.broadcasted_iota(jnp.int32, (LANES, LANES), 1)
    u_strict = (lane < laneT).astype(jnp.float32)
    row = jax.lax.broadcasted_iota(jnp.int32, (ROWS, ROWS), 0)
    rowT = jax.lax.broadcasted_iota(jnp.int32, (ROWS, ROWS), 1)
    l_strict = (rowT < row).astype(jnp.float32)

    def excl_cumsum(mask_f):
        within = _dot(mask_f, u_strict, ((1,), (0,)), _LO)
        rowsum = jnp.sum(mask_f, axis=1, keepdims=True)
        offs = _dot(l_strict, rowsum, ((1,), (0,)), _LO)
        return within + offs

    eq = (key == thr)
    eq_rank = excl_cumsum(eq.astype(jnp.float32))
    sel = (key > thr) | (eq & (eq_rank < extra))
    pos = excl_cumsum(sel.astype(jnp.float32))
    p_ref[:, :] = jnp.where(sel, pos, jnp.float32(D))


def _decode(sc, a1, a2, a3, a4, d1, d2, d3, d4, real):
    s = jnp.where(real, sc, NEG)
    dw = jnp.minimum(d3, CLIP)
    dh = jnp.minimum(d4, CLIP)
    widths = a3 - a1
    heights = a4 - a2
    ctr_x = a1 + 0.5 * widths
    ctr_y = a2 + 0.5 * heights
    pred_ctr_x = d1 * widths + ctr_x
    pred_ctr_y = d2 * heights + ctr_y
    pred_w = jnp.exp(dw) * widths
    pred_h = jnp.exp(dh) * heights
    x1 = jnp.clip(pred_ctr_x - 0.5 * pred_w, 0.0, IMG_W)
    y1 = jnp.clip(pred_ctr_y - 0.5 * pred_h, 0.0, IMG_H)
    x2 = jnp.clip(pred_ctr_x + 0.5 * pred_w, 0.0, IMG_W)
    y2 = jnp.clip(pred_ctr_y + 0.5 * pred_h, 0.0, IMG_H)
    valid = ((x2 - x1) >= MIN_SIZE) & ((y2 - y1) >= MIN_SIZE)
    ms = jnp.where(valid, s, NEG)
    area = (x2 - x1) * (y2 - y1)
    return x1, y1, x2, y2, area, ms, s


def _propose_kernel(p_ref, xt_ref, out_ref, mt_ref, supp_ref):
    f32 = jnp.float32
    ones11 = jnp.ones((1, 1), f32)
    slot_row = jax.lax.broadcasted_iota(jnp.int32, (1, D), 1).astype(f32)
    eye16 = (jax.lax.broadcasted_iota(jnp.int32, (16, 16), 0)
             == jax.lax.broadcasted_iota(jnp.int32, (16, 16), 1)).astype(f32)

    acc_t = jnp.zeros((16, D), dtype=f32)
    for c in range(N_PAD // CH):
        pc_row = p_ref[0:1, c * CH:(c + 1) * CH]
        pc_col = _dot(pc_row, ones11, ((0,), (0,)), _HI)
        onehot = (pc_col == slot_row).astype(f32)
        xc = xt_ref[:, c * CH:(c + 1) * CH]
        acc_t = acc_t + _dot(xc, onehot, ((1,), (0,)), _HI)
    acc = _dot(acc_t, eye16, ((0,), (0,)), _HI)

    i_col = jax.lax.broadcasted_iota(jnp.int32, (D, 1), 0)
    j_row = jax.lax.broadcasted_iota(jnp.int32, (1, D), 1)
    colv = _decode(acc[:, 0:1], acc[:, 1:2], acc[:, 2:3], acc[:, 3:4],
                   acc[:, 4:5], acc[:, 5:6], acc[:, 6:7], acc[:, 7:8],
                   acc[:, 8:9], i_col < K_SEL)
    rowv = _decode(acc_t[0:1, :], acc_t[1:2, :], acc_t[2:3, :], acc_t[3:4, :],
                   acc_t[4:5, :], acc_t[5:6, :], acc_t[6:7, :], acc_t[7:8, :],
                   acc_t[8:9, :], j_row < K_SEL)
    ms_c, s_c = colv[5], colv[6]
    ms_r, s_r = rowv[5], rowv[6]
    pos_c = i_col.astype(f32)
    pos_r = j_row.astype(f32)

    col8 = jax.lax.broadcasted_iota(jnp.int32, (1, 8), 1)
    P = jnp.zeros((D, 8), dtype=f32)
    for j, v in enumerate(colv + (pos_c,)):
        P = jnp.where(col8 == j, v, P)

    sp = jnp.zeros((D, 8), dtype=f32)
    for ib in range(D // B):
        sl = slice(ib * B, (ib + 1) * B)
        msb, sb, posb = ms_c[sl, :], s_c[sl, :], pos_c[sl, :]
        before = ((ms_r > msb)
                  | ((ms_r == msb)
                     & ((s_r > sb) | ((s_r == sb) & (pos_r < posb)))))
        rank_b = jnp.sum(before.astype(f32), axis=1, keepdims=True)
        perm_b = (rank_b == slot_row).astype(f32)
        sp = sp + _dot(perm_b, P[sl, :], ((0,), (0,)), _HI)
    eye8 = eye16[0:8, 0:8]
    spt = _dot(eye8, sp, ((1,), (1,)), _HI)

    for ib in range(D // B):
        sl = slice(ib * B, (ib + 1) * B)
        xx1 = jnp.maximum(sp[sl, 0:1], spt[0:1, :])
        yy1 = jnp.maximum(sp[sl, 1:2], spt[1:2, :])
        xx2 = jnp.minimum(sp[sl, 2:3], spt[2:3, :])
        yy2 = jnp.minimum(sp[sl, 3:4], spt[3:4, :])
        inter = jnp.maximum(xx2 - xx1, 0.0) * jnp.maximum(yy2 - yy1, 0.0)
        iou = inter / (sp[sl, 4:5] + spt[4:5, :] - inter + 1e-9)
        mt_ref[sl, :] = ((iou > NMS_THRESH)
                         & (i_col[sl, :] < j_row)).astype(jnp.bfloat16)

    bf16 = jnp.bfloat16
    for ib in range(D // B):
        sl = slice(ib * B, (ib + 1) * B)
        if ib == 0:
            ext = jnp.zeros((1, B), dtype=f32)
        else:
            keep_prev = (1.0 - supp_ref[0:1, 0:ib * B]).astype(bf16)
            hit_ext = _dot(keep_prev, mt_ref[0:ib * B, sl], ((1,), (0,)),
                           _LO)
            ext = (hit_ext > 0.0).astype(f32)
        mloc = mt_ref[sl, sl]

        def loc_cond(carry):
            _, done = carry
            return jnp.logical_not(done)

        def loc_body(carry, ext=ext, mloc=mloc):
            sloc, _ = carry
            keep = (1.0 - sloc).astype(bf16)
            hit = _dot(keep, mloc, ((1,), (0,)), _LO)
            new = jnp.where((hit > 0.0) | (ext > 0.0), 1.0, 0.0)
            return new, jnp.all(new == sloc)

        sloc, _ = jax.lax.while_loop(loc_cond, loc_body,
                                     (ext, jnp.bool_(False)))
        supp_ref[0:1, sl] = sloc

    supp = supp_ref[0:1, :]
    fs_r = jnp.where(supp > 0.0, NEG, spt[5:6, :])
    fs_c = _dot(fs_r, ones11, ((0,), (0,)), _HI)
    out_acc = jnp.zeros((OUT_PAD, 8), dtype=f32)
    rank_iota = jax.lax.broadcasted_iota(jnp.int32, (1, OUT_PAD), 1).astype(f32)
    for ib in range(D // B):
        sl = slice(ib * B, (ib + 1) * B)
        fb = ((fs_r > fs_c[sl, :])
              | ((fs_r == fs_c[sl, :]) & (pos_r < pos_c[sl, :])))
        frank_b = jnp.sum(fb.astype(f32), axis=1, keepdims=True)
        hb = (frank_b == rank_iota).astype(f32)
        out_acc = out_acc + _dot(hb, sp[sl, :], ((0,), (0,)), _HI)
    out_ref[:, :] = out_acc


@functools.partial(jax.jit, static_argnames=("interpret",))
def kernel(anchors, objectness, pred_bbox_delta, interpret=False):
    obj = objectness.astype(jnp.float32)
    obj_pad = jnp.full((N_PAD,), -jnp.inf, dtype=jnp.float32)
    obj_pad = jax.lax.dynamic_update_slice(obj_pad, obj, (0,))
    obj2d = obj_pad.reshape(ROWS, LANES)

    x = jnp.concatenate([obj[:, None], anchors.astype(jnp.float32),
                         pred_bbox_delta.astype(jnp.float32),
                         jnp.zeros((N_IN, 7), jnp.float32)], axis=1)
    x = jnp.pad(x, ((0, N_PAD - N_IN), (0, 0)))
    xt = x.T

    p2d = pl.pallas_call(
        _select_kernel,
        out_shape=jax.ShapeDtypeStruct((ROWS, LANES), jnp.float32),
        interpret=interpret,
    )(obj2d)
    p = p2d.reshape(1, N_PAD)

    out = pl.pallas_call(
        _propose_kernel,
        out_shape=jax.ShapeDtypeStruct((OUT_PAD, 8), jnp.float32),
        scratch_shapes=[pltpu.VMEM((D, D), jnp.bfloat16),
                        pltpu.VMEM((1, D), jnp.float32)],
        interpret=interpret,
    )(p, xt)
    return out[:OUT_K, :4]
